# baseline (speedup 1.0000x reference)
import jax
import jax.numpy as jnp
from jax.experimental import pallas as pl
from jax.experimental.pallas import tpu as pltpu

HIDDEN_DIM = 2048
NUM_EXPERTS = 64
TOP_K = 8
N_CANDIDATES = 32
BLOCK_ROWS = 1024


def _xor_partner_aligned(v, j):
    parts = []
    for base in range(0, v.shape[0], 2 * j):
        parts.append(v[base + j: base + 2 * j])
        parts.append(v[base: base + j])
    return jnp.concatenate(parts, axis=0)


def _router_block(x_ref, w_ref, probs_ref, vals_ref, idx_ref):
    E = NUM_EXPERTS
    x = x_ref[...]
    w = w_ref[...]
    logits = jax.lax.dot_general(
        x, w, (((1,), (0,)), ((), ())),
        preferred_element_type=jnp.float32,
        precision=jax.lax.Precision.DEFAULT)
    lt = logits.T
    gt = lt[:E]
    bt = lt[E:]

    m = jnp.max(gt, axis=0, keepdims=True)
    eg = jnp.exp(gt - m)
    pt = eg / jnp.sum(eg, axis=0, keepdims=True)
    probs_ref[...] = pt.T

    subl = jax.lax.broadcasted_iota(jnp.int32, (E, 1), 0)
    subl_f = subl.astype(jnp.float32)

    v = bt
    k = 2
    while k <= E:
        up = (subl & (k & (E - 1))) == 0
        j = k // 2
        while j >= 1:
            if j >= 8:
                vp = _xor_partner_aligned(v, j)
            else:
                vp = jnp.where((subl & j) == 0,
                               jnp.roll(v, -j, axis=0),
                               jnp.roll(v, j, axis=0))
            want_min = up == ((subl & j) == 0)
            v = jnp.where(want_min, jnp.minimum(v, vp), jnp.maximum(v, vp))
            j //= 2
        k *= 2
    t32 = v[N_CANDIDATES:N_CANDIDATES + 1]
    cand = bt >= t32

    masked = jnp.where(cand, pt, -1.0)
    vals_rows, idx_rows = [], []
    for _ in range(TOP_K):
        cur = jnp.max(masked, axis=0, keepdims=True)
        hit = masked == cur
        idxk = jnp.min(jnp.where(hit, subl_f, float(E)), axis=0,
                       keepdims=True)
        vals_rows.append(cur)
        idx_rows.append(idxk)
        masked = jnp.where(subl_f == idxk, -2.0, masked)
    vals8 = jnp.concatenate(vals_rows, axis=0)
    idx8 = jnp.concatenate(idx_rows, axis=0)

    vals8 = vals8 / jnp.sum(vals8, axis=0, keepdims=True)
    vals_ref[...] = vals8.T
    idx_ref[...] = idx8.T.astype(jnp.int32)


def kernel(hidden_states, W_gate, W_bvh):
    x = hidden_states.reshape(-1, HIDDEN_DIM)
    n = x.shape[0]
    w = jnp.concatenate([W_gate, W_bvh], axis=0).T
    b = BLOCK_ROWS
    grid = (n // b,)
    probs, vals, idx = pl.pallas_call(
        _router_block,
        grid=grid,
        in_specs=[
            pl.BlockSpec((b, HIDDEN_DIM), lambda i: (i, 0)),
            pl.BlockSpec((HIDDEN_DIM, 2 * NUM_EXPERTS), lambda i: (0, 0)),
        ],
        out_specs=[
            pl.BlockSpec((b, NUM_EXPERTS), lambda i: (i, 0)),
            pl.BlockSpec((b, TOP_K), lambda i: (i, 0)),
            pl.BlockSpec((b, TOP_K), lambda i: (i, 0)),
        ],
        out_shape=[
            jax.ShapeDtypeStruct((n, NUM_EXPERTS), jnp.float32),
            jax.ShapeDtypeStruct((n, TOP_K), jnp.float32),
            jax.ShapeDtypeStruct((n, TOP_K), jnp.int32),
        ],
        compiler_params=pltpu.CompilerParams(
            dimension_semantics=("arbitrary",)),
    )(x, w)
    return (probs, vals, idx)


# speedup vs baseline: 1.0303x; 1.0303x over previous
import jax
import jax.numpy as jnp
from jax.experimental import pallas as pl
from jax.experimental.pallas import tpu as pltpu

HIDDEN_DIM = 2048
NUM_EXPERTS = 64
TOP_K = 8
N_CANDIDATES = 32
BLOCK_ROWS = 2048


def _xor_partner_aligned(v, j):
    parts = []
    for base in range(0, v.shape[0], 2 * j):
        parts.append(v[base + j: base + 2 * j])
        parts.append(v[base: base + j])
    return jnp.concatenate(parts, axis=0)


def _router_block(x_ref, w_ref, probs_ref, vals_ref, idx_ref):
    E = NUM_EXPERTS
    x = x_ref[...]
    w = w_ref[...]
    logits = jax.lax.dot_general(
        x, w, (((1,), (0,)), ((), ())),
        preferred_element_type=jnp.float32,
        precision=jax.lax.Precision.DEFAULT)
    lt = logits.T
    gt = lt[:E]
    bt = lt[E:]

    m = jnp.max(gt, axis=0, keepdims=True)
    eg = jnp.exp(gt - m)
    pt = eg / jnp.sum(eg, axis=0, keepdims=True)
    probs_ref[...] = pt.T

    subl = jax.lax.broadcasted_iota(jnp.int32, (E, 1), 0)
    subl_f = subl.astype(jnp.float32)

    v = bt
    k = 2
    while k <= E:
        up = (subl & (k & (E - 1))) == 0
        j = k // 2
        while j >= 1:
            if j >= 8:
                vp = _xor_partner_aligned(v, j)
            else:
                vp = jnp.where((subl & j) == 0,
                               jnp.roll(v, -j, axis=0),
                               jnp.roll(v, j, axis=0))
            want_min = up == ((subl & j) == 0)
            v = jnp.where(want_min, jnp.minimum(v, vp), jnp.maximum(v, vp))
            j //= 2
        k *= 2
    t32 = v[N_CANDIDATES:N_CANDIDATES + 1]
    cand = bt >= t32

    masked = jnp.where(cand, pt, -1.0)
    vals_rows, idx_rows = [], []
    for _ in range(TOP_K):
        cur = jnp.max(masked, axis=0, keepdims=True)
        hit = masked == cur
        idxk = jnp.min(jnp.where(hit, subl_f, float(E)), axis=0,
                       keepdims=True)
        vals_rows.append(cur)
        idx_rows.append(idxk)
        masked = jnp.where(subl_f == idxk, -2.0, masked)
    vals8 = jnp.concatenate(vals_rows, axis=0)
    idx8 = jnp.concatenate(idx_rows, axis=0)

    vals8 = vals8 / jnp.sum(vals8, axis=0, keepdims=True)
    vals_ref[...] = vals8.T
    idx_ref[...] = idx8.T.astype(jnp.int32)


def kernel(hidden_states, W_gate, W_bvh):
    x = hidden_states.reshape(-1, HIDDEN_DIM)
    n = x.shape[0]
    w = jnp.concatenate([W_gate, W_bvh], axis=0).T
    b = BLOCK_ROWS
    grid = (n // b,)
    probs, vals, idx = pl.pallas_call(
        _router_block,
        grid=grid,
        in_specs=[
            pl.BlockSpec((b, HIDDEN_DIM), lambda i: (i, 0)),
            pl.BlockSpec((HIDDEN_DIM, 2 * NUM_EXPERTS), lambda i: (0, 0)),
        ],
        out_specs=[
            pl.BlockSpec((b, NUM_EXPERTS), lambda i: (i, 0)),
            pl.BlockSpec((b, TOP_K), lambda i: (i, 0)),
            pl.BlockSpec((b, TOP_K), lambda i: (i, 0)),
        ],
        out_shape=[
            jax.ShapeDtypeStruct((n, NUM_EXPERTS), jnp.float32),
            jax.ShapeDtypeStruct((n, TOP_K), jnp.float32),
            jax.ShapeDtypeStruct((n, TOP_K), jnp.int32),
        ],
        compiler_params=pltpu.CompilerParams(
            dimension_semantics=("parallel",)),
    )(x, w)
    return (probs, vals, idx)


# speedup vs baseline: 1.0442x; 1.0135x over previous
import jax
import jax.numpy as jnp
from jax.experimental import pallas as pl
from jax.experimental.pallas import tpu as pltpu

HIDDEN_DIM = 2048
NUM_EXPERTS = 64
TOP_K = 8
N_CANDIDATES = 32
HALF_ROWS = 1024
BLOCK_ROWS = 2 * HALF_ROWS


def _xor_partner_aligned(v, j):
    parts = []
    for base in range(0, v.shape[0], 2 * j):
        parts.append(v[base + j: base + 2 * j])
        parts.append(v[base: base + j])
    return jnp.concatenate(parts, axis=0)


def _route_half(x, w):
    E = NUM_EXPERTS
    logits = jax.lax.dot_general(
        x, w, (((1,), (0,)), ((), ())),
        preferred_element_type=jnp.float32,
        precision=jax.lax.Precision.DEFAULT)
    lt = logits.T
    gt = lt[:E]
    bt = lt[E:]

    m = jnp.max(gt, axis=0, keepdims=True)
    eg = jnp.exp(gt - m)
    pt = eg / jnp.sum(eg, axis=0, keepdims=True)

    subl = jax.lax.broadcasted_iota(jnp.int32, (E, 1), 0)
    subl_f = subl.astype(jnp.float32)

    v = bt
    k = 2
    while k <= E:
        up = (subl & (k & (E - 1))) == 0
        j = k // 2
        while j >= 1:
            if j >= 8:
                vp = _xor_partner_aligned(v, j)
            else:
                vp = jnp.where((subl & j) == 0,
                               jnp.roll(v, -j, axis=0),
                               jnp.roll(v, j, axis=0))
            want_min = up == ((subl & j) == 0)
            v = jnp.where(want_min, jnp.minimum(v, vp), jnp.maximum(v, vp))
            j //= 2
        k *= 2
    t32 = v[N_CANDIDATES:N_CANDIDATES + 1]
    cand = bt >= t32

    masked = jnp.where(cand, pt, -1.0)
    vals_rows, idx_rows = [], []
    for _ in range(TOP_K):
        cur = jnp.max(masked, axis=0, keepdims=True)
        hit = masked == cur
        idxk = jnp.min(jnp.where(hit, subl_f, float(E)), axis=0,
                       keepdims=True)
        vals_rows.append(cur)
        idx_rows.append(idxk)
        masked = jnp.where(subl_f == idxk, -2.0, masked)
    vals8 = jnp.concatenate(vals_rows, axis=0)
    idx8 = jnp.concatenate(idx_rows, axis=0)

    vals8 = vals8 / jnp.sum(vals8, axis=0, keepdims=True)
    return pt.T, vals8.T, idx8.T.astype(jnp.int32)


def _router_block(x1_ref, x2_ref, w_ref, probs_ref, vals_ref, idx_ref):
    w = w_ref[...]
    h = HALF_ROWS
    p1, v1, i1 = _route_half(x1_ref[...], w)
    probs_ref[0:h, :] = p1
    vals_ref[0:h, :] = v1
    idx_ref[0:h, :] = i1
    p2, v2, i2 = _route_half(x2_ref[...], w)
    probs_ref[h:2 * h, :] = p2
    vals_ref[h:2 * h, :] = v2
    idx_ref[h:2 * h, :] = i2


def kernel(hidden_states, W_gate, W_bvh):
    x = hidden_states.reshape(-1, HIDDEN_DIM)
    n = x.shape[0]
    w = jnp.concatenate([W_gate, W_bvh], axis=0).T
    grid = (n // BLOCK_ROWS,)
    probs, vals, idx = pl.pallas_call(
        _router_block,
        grid=grid,
        in_specs=[
            pl.BlockSpec((HALF_ROWS, HIDDEN_DIM), lambda i: (2 * i, 0)),
            pl.BlockSpec((HALF_ROWS, HIDDEN_DIM), lambda i: (2 * i + 1, 0)),
            pl.BlockSpec((HIDDEN_DIM, 2 * NUM_EXPERTS), lambda i: (0, 0)),
        ],
        out_specs=[
            pl.BlockSpec((BLOCK_ROWS, NUM_EXPERTS), lambda i: (i, 0)),
            pl.BlockSpec((BLOCK_ROWS, TOP_K), lambda i: (i, 0)),
            pl.BlockSpec((BLOCK_ROWS, TOP_K), lambda i: (i, 0)),
        ],
        out_shape=[
            jax.ShapeDtypeStruct((n, NUM_EXPERTS), jnp.float32),
            jax.ShapeDtypeStruct((n, TOP_K), jnp.float32),
            jax.ShapeDtypeStruct((n, TOP_K), jnp.int32),
        ],
        compiler_params=pltpu.CompilerParams(
            dimension_semantics=("arbitrary",)),
    )(x, x, w)
    return (probs, vals, idx)
